# baseline (speedup 1.0000x reference)
import jax
import jax.numpy as jnp
from jax import lax
from jax.experimental import pallas as pl
from jax.experimental.pallas import tpu as pltpu
from jax.experimental.pallas import tpu_sc as plsc

NUM_TOKENS = 1000
N_FIELDS = 26
N_NUMERIC = 13
BATCH = 1024
WIDTH = N_NUMERIC + N_FIELDS * NUM_TOKENS
L = 16
NW = 32
SLAB = 816
CH = 40
NFULL = 20
REM = SLAB - NFULL * CH
NJ = BATCH // L
UNROLL = 8
W31_FULL = 17
P32_LO = (NW - 1) * SLAB + W31_FULL * CH
P5_LO = P32_LO + 32
NB = 2


def _sc_body(xt_hbm, catt_hbm, mean_hbm, std_hbm, out_hbm,
             cat_v, m_s, s_s, stash, tail5, bufs_and_sems):
    bufs = bufs_and_sems[:NB]
    sems = bufs_and_sems[NB:NB + NB]
    tsem = bufs_and_sems[2 * NB]
    wid = lax.axis_index("s") * 2 + lax.axis_index("c")
    slab_lo = wid * SLAB

    f0 = jnp.maximum((slab_lo - N_NUMERIC) // NUM_TOKENS, 0)
    pltpu.sync_copy(catt_hbm.at[pl.ds(f0 * BATCH, 2 * BATCH)], cat_v)
    pltpu.sync_copy(mean_hbm, m_s)
    pltpu.sync_copy(std_hbm, s_s)

    zeros = jnp.zeros((L,), jnp.float32)
    ones = jnp.ones((L,), jnp.float32)
    iota = lax.iota(jnp.int32, L)

    def _zero(i, _):
        r = i >> 3
        for b in range(NB):
            for u in range(8):
                bufs[b][r, pl.ds(((i & 7) * 8 + u) * L, L)] = zeros
        return 0
    lax.fori_loop(0, CH * 8, _zero, 0)

    def _zero5(i, _):
        for u in range(8):
            tail5[i >> 3, pl.ds(((i & 7) * 8 + u) * L, L)] = zeros
        return 0
    lax.fori_loop(0, 5 * 8, _zero5, 0)

    def fld_of(lo):
        return jnp.maximum((lo - N_NUMERIC) // NUM_TOKENS, 0)

    def straddles(lo, hi):
        return fld_of(lo) != fld_of(hi - 1)

    def numeric(b, build):
        @pl.when(wid == 0)
        def _():
            if build:
                pltpu.sync_copy(xt_hbm, bufs[1].at[pl.ds(0, 16), :])

            def _row(c, _):
                cc = jnp.full((L,), c, jnp.int32)
                mc = plsc.load_gather(m_s, [cc])
                rc = ones / plsc.load_gather(s_s, [cc])

                def _col(jq, _):
                    for u in range(UNROLL):
                        j = jq * UNROLL + u
                        if build:
                            v = (bufs[1][c, pl.ds(j * L, L)] - mc) * rc
                        else:
                            v = zeros
                        bufs[b][c, pl.ds(j * L, L)] = v
                    return 0
                lax.fori_loop(0, NJ // UNROLL, _col, 0)
                return 0
            lax.fori_loop(0, N_NUMERIC, _row, 0)
            if build:
                def _rz(i, _):
                    for u in range(UNROLL):
                        bufs[1][i >> 4, pl.ds(((i & 15) * 4 + u) * L, L)] = \
                            zeros
                    return 0
                lax.fori_loop(0, 16 * 16, _rz, 0)

    def scan(buf, lo, hi, l, stash_plane, prev_plane):
        cbase = N_NUMERIC + (f0 + l) * NUM_TOKENS

        def _s(jq, _):
            for u in range(UNROLL):
                j = jq * UNROLL + u
                if prev_plane is not None:
                    plsc.store_scatter(
                        buf, [prev_plane[j], j * L + iota], zeros)
                c = cbase + cat_v[pl.ds(l * BATCH + j * L, L)]
                m = (c >= lo) & (c < hi)
                krow = jnp.where(m, c - lo, 0)
                if stash_plane is not None:
                    stash_plane[j] = krow
                plsc.store_scatter(buf, [krow, j * L + iota], ones, mask=m)
            return 0
        lax.fori_loop(0, NJ // UNROLL, _s, 0)

    class Plane:
        def __init__(self, b, p):
            self.b, self.p = b, p

        def __getitem__(self, j):
            return stash[self.b, self.p * NJ + j, :]

        def __setitem__(self, j, v):
            stash[self.b, self.p * NJ + j, :] = v

    def clear_plane(buf, plane):
        def _c(jq, _):
            for u in range(UNROLL):
                j = jq * UNROLL + u
                plsc.store_scatter(buf, [plane[j], j * L + iota], zeros)
            return 0
        lax.fori_loop(0, NJ // UNROLL, _c, 0)

    def chunk(b, lo, hi, prev_lo, prev_hi, buf=None):
        tgt = bufs[b] if buf is None else buf
        lcur = fld_of(lo) - f0
        p0, p1 = Plane(b, 0), Plane(b, 1)
        if prev_lo is not None:
            clear_plane(tgt, p0)

            @pl.when(straddles(prev_lo, prev_hi))
            def _():
                clear_plane(tgt, p1)

        scan(tgt, lo, hi, lcur, p0, None)

        @pl.when(straddles(lo, hi))
        def _():
            scan(tgt, lo, hi, lcur + 1, p1, None)

    def fire(b, lo):
        return pltpu.make_async_copy(
            bufs[b], out_hbm.at[pl.ds(lo, CH), :], sems[b])

    def step(k, b, after_wait=None):
        lo = slab_lo + k * CH
        fire(b, lo - NB * CH).wait()
        if after_wait is not None:
            after_wait()
        chunk(b, lo, lo + CH, lo - NB * CH, lo - NB * CH + CH)
        fire(b, lo).start()

    numeric(0, build=True)
    chunk(0, slab_lo, slab_lo + CH, None, None)
    fire(0, slab_lo).start()
    chunk(1, slab_lo + CH, slab_lo + 2 * CH, None, None)
    fire(1, slab_lo + CH).start()

    def _pair(k2, _):
        def _numclear():
            @pl.when(k2 == 1)
            def _():
                numeric(0, build=False)
        step(2 * k2, 0, after_wait=_numclear)
        step(2 * k2 + 1, 1)
        return 0
    lax.fori_loop(1, 8, _pair, 0)

    @pl.when(wid < NW - 1)
    def _():
        for k in range(16, NFULL):
            step(k, k & 1)
        lo = slab_lo + NFULL * CH
        plo = lo - NB * CH
        fire(0, plo).wait()
        chunk(0, lo, lo + REM, plo, plo + CH)
        rem = pltpu.make_async_copy(
            bufs[0].at[pl.ds(0, REM), :],
            out_hbm.at[pl.ds(lo, REM), :], sems[0])
        rem.start()
        fire(1, slab_lo + (NFULL - 1) * CH).wait()
        rem.wait()

    @pl.when(wid == NW - 1)
    def _():
        step(16, 0)
        plo = slab_lo + 15 * CH
        fire(1, plo).wait()
        chunk(1, P32_LO, P5_LO, plo, plo + CH)
        p32 = pltpu.make_async_copy(
            bufs[1].at[pl.ds(0, 32), :],
            out_hbm.at[pl.ds(P32_LO, 32), :], sems[1])
        p32.start()
        lcur = fld_of(P5_LO) - f0
        scan(tail5, P5_LO, WIDTH, lcur, None, None)
        p5 = pltpu.make_async_copy(
            tail5, out_hbm.at[pl.ds(P5_LO, WIDTH - P5_LO), :], tsem)
        p5.start()
        fire(0, slab_lo + 16 * CH).wait()
        p32.wait()
        p5.wait()


@jax.jit
def kernel(x_num, cat_idx, mean, std):
    xt_pad = jnp.pad(x_num.T, ((0, 3), (0, 0)))
    cat_t = jnp.pad(cat_idx.astype(jnp.int32).T, ((0, 6), (0, 0)))
    catt_flat = cat_t.reshape(-1)
    mean_b = jnp.pad(mean, (0, 3))
    std_b = jnp.pad(std, (0, 3), constant_values=1.0)

    mesh = plsc.VectorSubcoreMesh(core_axis_name="c", subcore_axis_name="s")
    f = pl.kernel(
        _sc_body,
        out_type=jax.ShapeDtypeStruct((WIDTH, BATCH), jnp.float32),
        mesh=mesh,
        compiler_params=pltpu.CompilerParams(needs_layout_passes=False,
                                             use_tc_tiling_on_sc=True),
        scratch_types=[
            pltpu.VMEM((2 * BATCH,), jnp.int32),
            pltpu.VMEM((L,), jnp.float32),
            pltpu.VMEM((L,), jnp.float32),
            pltpu.VMEM((NB, 2 * NJ, L), jnp.int32),
            pltpu.VMEM((5, BATCH), jnp.float32),
            [pltpu.VMEM((CH, BATCH), jnp.float32) for _ in range(NB)]
            + [pltpu.SemaphoreType.DMA for _ in range(NB)]
            + [pltpu.SemaphoreType.DMA],
        ],
    )
    return f(xt_pad, catt_flat, mean_b, std_b).T


# speedup vs baseline: 1.1318x; 1.1318x over previous
import jax
import jax.numpy as jnp
from jax import lax
from jax.experimental import pallas as pl
from jax.experimental.pallas import tpu as pltpu
from jax.experimental.pallas import tpu_sc as plsc

NUM_TOKENS = 1000
N_FIELDS = 26
N_NUMERIC = 13
BATCH = 1024
WIDTH = N_NUMERIC + N_FIELDS * NUM_TOKENS
L = 16
NW = 32
SLAB = 816
CH = 40
NFULL = 20
REM = SLAB - NFULL * CH
NJ = BATCH // L
UNROLL = 4
W31_FULL = 17
P32_LO = (NW - 1) * SLAB + W31_FULL * CH
P5_LO = P32_LO + 32
NB = 2


def _sc_body(xt_hbm, catt_hbm, mean_hbm, std_hbm, out_hbm,
             cat_v, m_s, s_s, stash, tail5, bufs_and_sems):
    bufs = bufs_and_sems[:NB]
    sems = bufs_and_sems[NB:NB + NB]
    tsem = bufs_and_sems[2 * NB]
    wid = lax.axis_index("s") * 2 + lax.axis_index("c")
    slab_lo = wid * SLAB

    f0 = jnp.maximum((slab_lo - N_NUMERIC) // NUM_TOKENS, 0)
    pltpu.sync_copy(catt_hbm.at[pl.ds(f0 * BATCH, 2 * BATCH)], cat_v)
    pltpu.sync_copy(mean_hbm, m_s)
    pltpu.sync_copy(std_hbm, s_s)

    zeros = jnp.zeros((L,), jnp.float32)
    ones = jnp.ones((L,), jnp.float32)
    iota = lax.iota(jnp.int32, L)

    def _zero(i, _):
        r = i >> 3
        for b in range(NB):
            for u in range(8):
                bufs[b][r, pl.ds(((i & 7) * 8 + u) * L, L)] = zeros
        return 0
    lax.fori_loop(0, CH * 8, _zero, 0)

    def _zero5(i, _):
        for u in range(8):
            tail5[i >> 3, pl.ds(((i & 7) * 8 + u) * L, L)] = zeros
        return 0
    lax.fori_loop(0, 5 * 8, _zero5, 0)

    def fld_of(lo):
        return jnp.maximum((lo - N_NUMERIC) // NUM_TOKENS, 0)

    def straddles(lo, hi):
        return fld_of(lo) != fld_of(hi - 1)

    def numeric(b, build):
        @pl.when(wid == 0)
        def _():
            if build:
                pltpu.sync_copy(xt_hbm, bufs[1].at[pl.ds(0, 16), :])

            def _row(c, _):
                cc = jnp.full((L,), c, jnp.int32)
                mc = plsc.load_gather(m_s, [cc])
                rc = ones / plsc.load_gather(s_s, [cc])

                def _col(jq, _):
                    for u in range(UNROLL):
                        j = jq * UNROLL + u
                        if build:
                            v = (bufs[1][c, pl.ds(j * L, L)] - mc) * rc
                        else:
                            v = zeros
                        bufs[b][c, pl.ds(j * L, L)] = v
                    return 0
                lax.fori_loop(0, NJ // UNROLL, _col, 0)
                return 0
            lax.fori_loop(0, N_NUMERIC, _row, 0)
            if build:
                def _rz(i, _):
                    for u in range(UNROLL):
                        bufs[1][i >> 4, pl.ds(((i & 15) * 4 + u) * L, L)] = \
                            zeros
                    return 0
                lax.fori_loop(0, 16 * 16, _rz, 0)

    def scan(buf, lo, hi, l, stash_plane, prev_plane):
        cbase = N_NUMERIC + (f0 + l) * NUM_TOKENS

        def _s(jq, _):
            for u in range(UNROLL):
                j = jq * UNROLL + u
                if prev_plane is not None:
                    plsc.store_scatter(
                        buf, [prev_plane[j], j * L + iota], zeros)
                c = cbase + cat_v[pl.ds(l * BATCH + j * L, L)]
                m = (c >= lo) & (c < hi)
                krow = jnp.where(m, c - lo, 0)
                if stash_plane is not None:
                    stash_plane[j] = krow
                plsc.store_scatter(buf, [krow, j * L + iota], ones, mask=m)
            return 0
        lax.fori_loop(0, NJ // UNROLL, _s, 0)

    class Plane:
        def __init__(self, b, p):
            self.b, self.p = b, p

        def __getitem__(self, j):
            return stash[self.b, self.p * NJ + j, :]

        def __setitem__(self, j, v):
            stash[self.b, self.p * NJ + j, :] = v

    def clear_plane(buf, plane):
        def _c(jq, _):
            for u in range(UNROLL):
                j = jq * UNROLL + u
                plsc.store_scatter(buf, [plane[j], j * L + iota], zeros)
            return 0
        lax.fori_loop(0, NJ // UNROLL, _c, 0)

    def chunk(b, lo, hi, prev_lo, prev_hi, buf=None):
        tgt = bufs[b] if buf is None else buf
        lcur = fld_of(lo) - f0
        p0, p1 = Plane(b, 0), Plane(b, 1)
        if prev_lo is not None:
            clear_plane(tgt, p0)

            @pl.when(straddles(prev_lo, prev_hi))
            def _():
                clear_plane(tgt, p1)

        scan(tgt, lo, hi, lcur, p0, None)

        @pl.when(straddles(lo, hi))
        def _():
            scan(tgt, lo, hi, lcur + 1, p1, None)

    def fire(b, lo):
        return pltpu.make_async_copy(
            bufs[b], out_hbm.at[pl.ds(lo, CH), :], sems[b])

    def step(k, b, after_wait=None):
        lo = slab_lo + k * CH
        fire(b, lo - NB * CH).wait()
        if after_wait is not None:
            after_wait()
        chunk(b, lo, lo + CH, lo - NB * CH, lo - NB * CH + CH)
        fire(b, lo).start()

    numeric(0, build=True)
    chunk(0, slab_lo, slab_lo + CH, None, None)
    fire(0, slab_lo).start()
    chunk(1, slab_lo + CH, slab_lo + 2 * CH, None, None)
    fire(1, slab_lo + CH).start()

    def _pair(k2, _):
        def _numclear():
            @pl.when(k2 == 1)
            def _():
                numeric(0, build=False)
        step(2 * k2, 0, after_wait=_numclear)
        step(2 * k2 + 1, 1)
        return 0
    lax.fori_loop(1, 8, _pair, 0)

    @pl.when(wid < NW - 1)
    def _():
        for k in range(16, NFULL):
            step(k, k & 1)
        lo = slab_lo + NFULL * CH
        plo = lo - NB * CH
        fire(0, plo).wait()
        chunk(0, lo, lo + REM, plo, plo + CH)
        rem = pltpu.make_async_copy(
            bufs[0].at[pl.ds(0, REM), :],
            out_hbm.at[pl.ds(lo, REM), :], sems[0])
        rem.start()
        fire(1, slab_lo + (NFULL - 1) * CH).wait()
        rem.wait()

    @pl.when(wid == NW - 1)
    def _():
        step(16, 0)
        plo = slab_lo + 15 * CH
        fire(1, plo).wait()
        chunk(1, P32_LO, P5_LO, plo, plo + CH)
        p32 = pltpu.make_async_copy(
            bufs[1].at[pl.ds(0, 32), :],
            out_hbm.at[pl.ds(P32_LO, 32), :], sems[1])
        p32.start()
        lcur = fld_of(P5_LO) - f0
        scan(tail5, P5_LO, WIDTH, lcur, None, None)
        p5 = pltpu.make_async_copy(
            tail5, out_hbm.at[pl.ds(P5_LO, WIDTH - P5_LO), :], tsem)
        p5.start()
        fire(0, slab_lo + 16 * CH).wait()
        p32.wait()
        p5.wait()


@jax.jit
def kernel(x_num, cat_idx, mean, std):
    xt_pad = jnp.pad(x_num.T, ((0, 3), (0, 0)))
    cat_t = jnp.pad(cat_idx.astype(jnp.int32).T, ((0, 6), (0, 0)))
    catt_flat = cat_t.reshape(-1)
    mean_b = jnp.pad(mean, (0, 3))
    std_b = jnp.pad(std, (0, 3), constant_values=1.0)

    mesh = plsc.VectorSubcoreMesh(core_axis_name="c", subcore_axis_name="s")
    f = pl.kernel(
        _sc_body,
        out_type=jax.ShapeDtypeStruct((WIDTH, BATCH), jnp.float32),
        mesh=mesh,
        compiler_params=pltpu.CompilerParams(needs_layout_passes=False,
                                             use_tc_tiling_on_sc=True),
        scratch_types=[
            pltpu.VMEM((2 * BATCH,), jnp.int32),
            pltpu.VMEM((L,), jnp.float32),
            pltpu.VMEM((L,), jnp.float32),
            pltpu.VMEM((NB, 2 * NJ, L), jnp.int32),
            pltpu.VMEM((5, BATCH), jnp.float32),
            [pltpu.VMEM((CH, BATCH), jnp.float32) for _ in range(NB)]
            + [pltpu.SemaphoreType.DMA for _ in range(NB)]
            + [pltpu.SemaphoreType.DMA],
        ],
    )
    return f(xt_pad, catt_flat, mean_b, std_b).T


# speedup vs baseline: 1.1328x; 1.0009x over previous
import jax
import jax.numpy as jnp
from jax import lax
from jax.experimental import pallas as pl
from jax.experimental.pallas import tpu as pltpu
from jax.experimental.pallas import tpu_sc as plsc

NUM_TOKENS = 1000
N_FIELDS = 26
N_NUMERIC = 13
BATCH = 1024
WIDTH = N_NUMERIC + N_FIELDS * NUM_TOKENS
L = 16
NW = 32
SLAB = 816
CH = 40
NFULL = 20
REM = SLAB - NFULL * CH
NJ = BATCH // L
UNROLL = 4
W31_FULL = 17
P32_LO = (NW - 1) * SLAB + W31_FULL * CH
P5_LO = P32_LO + 32
NB = 2


def _sc_body(xt_hbm, catt_hbm, mean_hbm, std_hbm, out_hbm,
             cat_v, m_s, s_s, stash, tail5, bufs_and_sems):
    bufs = bufs_and_sems[:NB]
    sems = bufs_and_sems[NB:NB + NB]
    tsem = bufs_and_sems[2 * NB]
    wid = lax.axis_index("s") * 2 + lax.axis_index("c")
    slab_lo = wid * SLAB

    f0 = jnp.maximum((slab_lo - N_NUMERIC) // NUM_TOKENS, 0)
    pltpu.sync_copy(catt_hbm.at[pl.ds(f0 * BATCH, 2 * BATCH)], cat_v)
    pltpu.sync_copy(mean_hbm, m_s)
    pltpu.sync_copy(std_hbm, s_s)

    zeros = jnp.zeros((L,), jnp.float32)
    ones = jnp.ones((L,), jnp.float32)
    iota = lax.iota(jnp.int32, L)

    def _zero(i, _):
        r = i >> 3
        for b in range(NB):
            for u in range(8):
                bufs[b][r, pl.ds(((i & 7) * 8 + u) * L, L)] = zeros
        return 0
    lax.fori_loop(0, CH * 8, _zero, 0)

    def _zero5(i, _):
        for u in range(8):
            tail5[i >> 3, pl.ds(((i & 7) * 8 + u) * L, L)] = zeros
        return 0
    lax.fori_loop(0, 5 * 8, _zero5, 0)

    def fld_of(lo):
        return jnp.maximum((lo - N_NUMERIC) // NUM_TOKENS, 0)

    def straddles(lo, hi):
        return fld_of(lo) != fld_of(hi - 1)

    def numeric(b, build):
        @pl.when(wid == 0)
        def _():
            if build:
                pltpu.sync_copy(xt_hbm, bufs[1].at[pl.ds(0, 16), :])

            def _row(c, _):
                cc = jnp.full((L,), c, jnp.int32)
                mc = plsc.load_gather(m_s, [cc])
                rc = ones / plsc.load_gather(s_s, [cc])

                def _col(jq, _):
                    for u in range(UNROLL):
                        j = jq * UNROLL + u
                        if build:
                            v = (bufs[1][c, pl.ds(j * L, L)] - mc) * rc
                        else:
                            v = zeros
                        bufs[b][c, pl.ds(j * L, L)] = v
                    return 0
                lax.fori_loop(0, NJ // UNROLL, _col, 0)
                return 0
            lax.fori_loop(0, N_NUMERIC, _row, 0)
            if build:
                def _rz(i, _):
                    for u in range(UNROLL):
                        bufs[1][i >> 4, pl.ds(((i & 15) * 4 + u) * L, L)] = \
                            zeros
                    return 0
                lax.fori_loop(0, 16 * 16, _rz, 0)

    def scan(buf, lo, hi, l, stash_plane, prev_plane):
        cbase = N_NUMERIC + (f0 + l) * NUM_TOKENS

        width = plsc.bitcast(jnp.full((L,), hi - lo, jnp.int32), jnp.uint32)

        def _s(jq, _):
            for u in range(UNROLL):
                j = jq * UNROLL + u
                if prev_plane is not None:
                    plsc.store_scatter(
                        buf, [prev_plane[j], j * L + iota], zeros)
                ku = cbase + cat_v[pl.ds(l * BATCH + j * L, L)] - lo
                m = plsc.bitcast(ku, jnp.uint32) < width
                krow = jnp.where(m, ku, 0)
                if stash_plane is not None:
                    stash_plane[j] = krow
                plsc.store_scatter(buf, [krow, j * L + iota], ones, mask=m)
            return 0
        lax.fori_loop(0, NJ // UNROLL, _s, 0)

    class Plane:
        def __init__(self, b, p):
            self.b, self.p = b, p

        def __getitem__(self, j):
            return stash[self.b, self.p * NJ + j, :]

        def __setitem__(self, j, v):
            stash[self.b, self.p * NJ + j, :] = v

    def clear_plane(buf, plane):
        def _c(jq, _):
            for u in range(UNROLL):
                j = jq * UNROLL + u
                plsc.store_scatter(buf, [plane[j], j * L + iota], zeros)
            return 0
        lax.fori_loop(0, NJ // UNROLL, _c, 0)

    def chunk(b, lo, hi, prev_lo, prev_hi, buf=None):
        tgt = bufs[b] if buf is None else buf
        lcur = fld_of(lo) - f0
        p0, p1 = Plane(b, 0), Plane(b, 1)
        if prev_lo is not None:
            clear_plane(tgt, p0)

            @pl.when(straddles(prev_lo, prev_hi))
            def _():
                clear_plane(tgt, p1)

        scan(tgt, lo, hi, lcur, p0, None)

        @pl.when(straddles(lo, hi))
        def _():
            scan(tgt, lo, hi, lcur + 1, p1, None)

    def fire(b, lo):
        return pltpu.make_async_copy(
            bufs[b], out_hbm.at[pl.ds(lo, CH), :], sems[b])

    def step(k, b, after_wait=None):
        lo = slab_lo + k * CH
        fire(b, lo - NB * CH).wait()
        if after_wait is not None:
            after_wait()
        chunk(b, lo, lo + CH, lo - NB * CH, lo - NB * CH + CH)
        fire(b, lo).start()

    numeric(0, build=True)
    chunk(0, slab_lo, slab_lo + CH, None, None)
    fire(0, slab_lo).start()
    chunk(1, slab_lo + CH, slab_lo + 2 * CH, None, None)
    fire(1, slab_lo + CH).start()

    def _pair(k2, _):
        def _numclear():
            @pl.when(k2 == 1)
            def _():
                numeric(0, build=False)
        step(2 * k2, 0, after_wait=_numclear)
        step(2 * k2 + 1, 1)
        return 0
    lax.fori_loop(1, 8, _pair, 0)

    @pl.when(wid < NW - 1)
    def _():
        for k in range(16, NFULL):
            step(k, k & 1)
        lo = slab_lo + NFULL * CH
        plo = lo - NB * CH
        fire(0, plo).wait()
        chunk(0, lo, lo + REM, plo, plo + CH)
        rem = pltpu.make_async_copy(
            bufs[0].at[pl.ds(0, REM), :],
            out_hbm.at[pl.ds(lo, REM), :], sems[0])
        rem.start()
        fire(1, slab_lo + (NFULL - 1) * CH).wait()
        rem.wait()

    @pl.when(wid == NW - 1)
    def _():
        step(16, 0)
        plo = slab_lo + 15 * CH
        fire(1, plo).wait()
        chunk(1, P32_LO, P5_LO, plo, plo + CH)
        p32 = pltpu.make_async_copy(
            bufs[1].at[pl.ds(0, 32), :],
            out_hbm.at[pl.ds(P32_LO, 32), :], sems[1])
        p32.start()
        lcur = fld_of(P5_LO) - f0
        scan(tail5, P5_LO, WIDTH, lcur, None, None)
        p5 = pltpu.make_async_copy(
            tail5, out_hbm.at[pl.ds(P5_LO, WIDTH - P5_LO), :], tsem)
        p5.start()
        fire(0, slab_lo + 16 * CH).wait()
        p32.wait()
        p5.wait()


@jax.jit
def kernel(x_num, cat_idx, mean, std):
    xt_pad = jnp.pad(x_num.T, ((0, 3), (0, 0)))
    cat_t = jnp.pad(cat_idx.astype(jnp.int32).T, ((0, 6), (0, 0)))
    catt_flat = cat_t.reshape(-1)
    mean_b = jnp.pad(mean, (0, 3))
    std_b = jnp.pad(std, (0, 3), constant_values=1.0)

    mesh = plsc.VectorSubcoreMesh(core_axis_name="c", subcore_axis_name="s")
    f = pl.kernel(
        _sc_body,
        out_type=jax.ShapeDtypeStruct((WIDTH, BATCH), jnp.float32),
        mesh=mesh,
        compiler_params=pltpu.CompilerParams(needs_layout_passes=False,
                                             use_tc_tiling_on_sc=True),
        scratch_types=[
            pltpu.VMEM((2 * BATCH,), jnp.int32),
            pltpu.VMEM((L,), jnp.float32),
            pltpu.VMEM((L,), jnp.float32),
            pltpu.VMEM((NB, 2 * NJ, L), jnp.int32),
            pltpu.VMEM((5, BATCH), jnp.float32),
            [pltpu.VMEM((CH, BATCH), jnp.float32) for _ in range(NB)]
            + [pltpu.SemaphoreType.DMA for _ in range(NB)]
            + [pltpu.SemaphoreType.DMA],
        ],
    )
    return f(xt_pad, catt_flat, mean_b, std_b).T
